# Initial kernel scaffold; baseline (speedup 1.0000x reference)
import jax
import jax.numpy as jnp
from jax.experimental import pallas as pl


def kernel(tokens, emb, W1, U1, b1, W2, U2, b2, fc1_W, fc1_b, fc2_W, fc2_b):
    raise NotImplementedError("write your pallas kernel here")


# speedup vs baseline: 1.2539x; 1.2539x over previous
import functools

import jax
import jax.numpy as jnp
from jax import lax
from jax.experimental import pallas as pl
from jax.experimental.pallas import tpu as pltpu
from jax.experimental.pallas import tpu_sc as plsc

VOCAB = 1000000
EMB = 64
UNITS = 64
DENSE = 250
B = 1024
L = 200

NC = 2
NS = 16
NW = NC * NS
TOTAL = B * L
CHUNK = 128
ROWS_PER_W = TOTAL // NW
CHUNKS_PER_W = ROWS_PER_W // CHUNK


def _sc_gather_body(emb_hbm, tok_hbm, x_hbm, idx_v, rows_v, sem):
    wid = lax.axis_index("s") * NC + lax.axis_index("c")
    base = wid * CHUNKS_PER_W
    pltpu.sync_copy(tok_hbm.at[wid], idx_v)

    def step(c, carry):
        pltpu.async_copy(emb_hbm.at[idx_v.at[c]], rows_v, sem).wait()
        pltpu.sync_copy(rows_v, x_hbm.at[pl.ds((base + c) * CHUNK, CHUNK)])
        return carry

    lax.fori_loop(0, CHUNKS_PER_W, step, 0)


def _sc_gather(emb, tok2d):
    mesh = plsc.VectorSubcoreMesh(core_axis_name="c", subcore_axis_name="s")
    fn = pl.kernel(
        _sc_gather_body,
        out_type=jax.ShapeDtypeStruct((TOTAL, EMB), jnp.float32),
        mesh=mesh,
        scratch_types=[
            pltpu.VMEM((CHUNKS_PER_W, CHUNK), jnp.int32),
            pltpu.VMEM((CHUNK, EMB), jnp.float32),
            pltpu.SemaphoreType.DMA,
        ],
        compiler_params=pltpu.CompilerParams(use_tc_tiling_on_sc=False),
    )
    return fn(emb, tok2d)


def _lstm_body(x_ref, W1_ref, U1_ref, b1_ref, W2_ref, U2_ref, b2_ref,
               fc1W_ref, fc1b_ref, fc2Wt_ref, fc2b_ref, out_ref,
               h1, c1, h2, c2):
    t = pl.program_id(0)

    @pl.when(t == 0)
    def _():
        h1[...] = jnp.zeros_like(h1)
        c1[...] = jnp.zeros_like(c1)
        h2[...] = jnp.zeros_like(h2)
        c2[...] = jnp.zeros_like(c2)

    def dot(a, b):
        return lax.dot_general(a, b, (((1,), (0,)), ((), ())),
                               preferred_element_type=jnp.float32)

    def gates(z, c_prev):
        i = jax.nn.sigmoid(z[:, 0 * UNITS:1 * UNITS])
        f = jax.nn.sigmoid(z[:, 1 * UNITS:2 * UNITS])
        g = jnp.tanh(z[:, 2 * UNITS:3 * UNITS])
        o = jax.nn.sigmoid(z[:, 3 * UNITS:4 * UNITS])
        c_new = f * c_prev + i * g
        h_new = o * jnp.tanh(c_new)
        return h_new, c_new

    x_t = x_ref[0]
    z1 = dot(x_t, W1_ref[...]) + dot(h1[...], U1_ref[...]) + b1_ref[...]
    h1_new, c1_new = gates(z1, c1[...])
    h1[...] = h1_new
    c1[...] = c1_new

    z2 = dot(h1_new, W2_ref[...]) + dot(h2[...], U2_ref[...]) + b2_ref[...]
    h2_new, c2_new = gates(z2, c2[...])
    h2[...] = h2_new
    c2[...] = c2_new

    @pl.when(t == L - 1)
    def _():
        d = jnp.tanh(dot(h2_new, fc1W_ref[...]) + fc1b_ref[...])
        o = jnp.sum(d * fc2Wt_ref[...], axis=1, keepdims=True) + fc2b_ref[...]
        out_ref[...] = jax.nn.sigmoid(o)


def _lstm_head(x, W1, U1, b1, W2, U2, b2, fc1_W, fc1_b, fc2_Wt, fc2_b):
    full = lambda shape: pl.BlockSpec(shape, lambda t: (0,) * len(shape))
    return pl.pallas_call(
        _lstm_body,
        grid=(L,),
        in_specs=[
            pl.BlockSpec((1, B, EMB), lambda t: (t, 0, 0)),
            full((EMB, 4 * UNITS)),
            full((UNITS, 4 * UNITS)),
            full((1, 4 * UNITS)),
            full((UNITS, 4 * UNITS)),
            full((UNITS, 4 * UNITS)),
            full((1, 4 * UNITS)),
            full((UNITS, DENSE)),
            full((1, DENSE)),
            full((1, DENSE)),
            full((1, 1)),
        ],
        out_specs=pl.BlockSpec((B, 1), lambda t: (0, 0)),
        out_shape=jax.ShapeDtypeStruct((B, 1), jnp.float32),
        scratch_shapes=[pltpu.VMEM((B, UNITS), jnp.float32)] * 4,
    )(x, W1, U1, b1, W2, U2, b2, fc1_W, fc1_b, fc2_Wt, fc2_b)


def kernel(tokens, emb, W1, U1, b1, W2, U2, b2, fc1_W, fc1_b, fc2_W, fc2_b):
    tok2d = jnp.transpose(tokens).astype(jnp.int32).reshape(NW, CHUNKS_PER_W, CHUNK)
    x = _sc_gather(emb, tok2d).reshape(L, B, EMB)
    out = _lstm_head(
        x, W1, U1, b1.reshape(1, -1), W2, U2, b2.reshape(1, -1),
        fc1_W, fc1_b.reshape(1, -1), fc2_W.reshape(1, -1), fc2_b.reshape(1, 1),
    )
    return out


# speedup vs baseline: 1.6190x; 1.2912x over previous
import jax
import jax.numpy as jnp
from jax import lax
from jax.experimental import pallas as pl
from jax.experimental.pallas import tpu as pltpu
from jax.experimental.pallas import tpu_sc as plsc

VOCAB = 1000000
EMB = 64
XPAD = 128
UNITS = 64
DENSE = 250
B = 1024
L = 200

NC = 2
NS = 16
NW = NC * NS
TOTAL = B * L
CHUNK = 128
ROWS_PER_W = TOTAL // NW
CHUNKS_PER_W = ROWS_PER_W // CHUNK

TBLK = 8
NGRID = L // TBLK


def _sc_gather_body(emb_hbm, tok_hbm, x_hbm, idx_v, rows_v, sem):
    wid = lax.axis_index("s") * NC + lax.axis_index("c")
    base = wid * CHUNKS_PER_W
    pltpu.sync_copy(tok_hbm.at[wid], idx_v)

    def step(c, carry):
        pltpu.async_copy(emb_hbm.at[idx_v.at[c]], rows_v, sem).wait()
        pltpu.sync_copy(rows_v,
                        x_hbm.at[pl.ds((base + c) * CHUNK, CHUNK), pl.ds(0, EMB)])
        return carry

    lax.fori_loop(0, CHUNKS_PER_W, step, 0)


def _sc_gather(emb, tok3d):
    mesh = plsc.VectorSubcoreMesh(core_axis_name="c", subcore_axis_name="s")
    fn = pl.kernel(
        _sc_gather_body,
        out_type=jax.ShapeDtypeStruct((TOTAL, XPAD), jnp.float32),
        mesh=mesh,
        scratch_types=[
            pltpu.VMEM((CHUNKS_PER_W, CHUNK), jnp.int32),
            pltpu.VMEM((CHUNK, EMB), jnp.float32),
            pltpu.SemaphoreType.DMA,
        ],
        compiler_params=pltpu.CompilerParams(use_tc_tiling_on_sc=False),
    )
    return fn(emb, tok3d)


def _sigm(x):
    return 0.5 * jnp.tanh(0.5 * x) + 0.5


def _lstm_body(x_ref, W1_ref, U1_ref, b1_ref, W2_ref, U2_ref, b2_ref,
               fc1W_ref, fc1b_ref, fc2Wt_ref, fc2b_ref, out_ref,
               h1_s, c1_s, h2_s, c2_s):
    g = pl.program_id(0)

    @pl.when(g == 0)
    def _():
        h1_s[...] = jnp.zeros_like(h1_s)
        c1_s[...] = jnp.zeros_like(c1_s)
        h2_s[...] = jnp.zeros_like(h2_s)
        c2_s[...] = jnp.zeros_like(c2_s)

    def dot(a, b):
        return lax.dot_general(a, b, (((1,), (0,)), ((), ())),
                               preferred_element_type=jnp.float32)

    def cell(z, c_prev):
        i = _sigm(z[:, 0 * UNITS:1 * UNITS])
        f = _sigm(z[:, 1 * UNITS:2 * UNITS])
        gg = jnp.tanh(z[:, 2 * UNITS:3 * UNITS])
        o = _sigm(z[:, 3 * UNITS:4 * UNITS])
        c_new = f * c_prev + i * gg
        h_new = o * jnp.tanh(c_new)
        return h_new, c_new

    W1 = W1_ref[...]
    U1 = U1_ref[...]
    b1 = b1_ref[...]
    W2 = W2_ref[...]
    U2 = U2_ref[...]
    b2 = b2_ref[...]

    h1, c1 = h1_s[...], c1_s[...]
    h2, c2 = h2_s[...], c2_s[...]
    for j in range(TBLK):
        x_t = x_ref[j, :, 0:EMB]
        h1, c1 = cell(dot(x_t, W1) + dot(h1, U1) + b1, c1)
        h2, c2 = cell(dot(h1, W2) + dot(h2, U2) + b2, c2)
    h1_s[...], c1_s[...] = h1, c1
    h2_s[...], c2_s[...] = h2, c2

    @pl.when(g == NGRID - 1)
    def _():
        d = jnp.tanh(dot(h2, fc1W_ref[...]) + fc1b_ref[...])
        o = jnp.sum(d * fc2Wt_ref[...], axis=1, keepdims=True) + fc2b_ref[...]
        out_ref[...] = _sigm(o)


def _lstm_head(x, W1, U1, b1, W2, U2, b2, fc1_W, fc1_b, fc2_Wt, fc2_b):
    full = lambda shape: pl.BlockSpec(shape, lambda g: (0,) * len(shape))
    return pl.pallas_call(
        _lstm_body,
        grid=(NGRID,),
        in_specs=[
            pl.BlockSpec((TBLK, B, XPAD), lambda g: (g, 0, 0)),
            full((EMB, 4 * UNITS)),
            full((UNITS, 4 * UNITS)),
            full((1, 4 * UNITS)),
            full((UNITS, 4 * UNITS)),
            full((UNITS, 4 * UNITS)),
            full((1, 4 * UNITS)),
            full((UNITS, DENSE)),
            full((1, DENSE)),
            full((1, DENSE)),
            full((1, 1)),
        ],
        out_specs=pl.BlockSpec((B, 1), lambda g: (0, 0)),
        out_shape=jax.ShapeDtypeStruct((B, 1), jnp.float32),
        scratch_shapes=[pltpu.VMEM((B, UNITS), jnp.float32)] * 4,
    )(x, W1, U1, b1, W2, U2, b2, fc1_W, fc1_b, fc2_Wt, fc2_b)


def kernel(tokens, emb, W1, U1, b1, W2, U2, b2, fc1_W, fc1_b, fc2_W, fc2_b):
    tok3d = jnp.transpose(tokens).astype(jnp.int32).reshape(NW, CHUNKS_PER_W, CHUNK)
    x = _sc_gather(emb, tok3d).reshape(L, B, XPAD)
    out = _lstm_head(
        x, W1, U1, b1.reshape(1, -1), W2, U2, b2.reshape(1, -1),
        fc1_W, fc1_b.reshape(1, -1), fc2_W.reshape(1, -1), fc2_b.reshape(1, 1),
    )
    return out


# speedup vs baseline: 1.6440x; 1.0154x over previous
import jax
import jax.numpy as jnp
from jax import lax
from jax.experimental import pallas as pl
from jax.experimental.pallas import tpu as pltpu
from jax.experimental.pallas import tpu_sc as plsc
from jax.experimental import layout as jexp_layout

VOCAB = 1000000
EMB = 64
XPAD = 128
UNITS = 64
DENSE = 250
B = 1024
L = 200

NC = 2
NS = 16
NW = NC * NS
TOTAL = B * L
CHUNK = 128
ROWS_PER_W = TOTAL // NW
CHUNKS_PER_W = ROWS_PER_W // CHUNK

TBLK = 8
NGRID = L // TBLK


def _sc_gather_body(emb_hbm, tok_hbm, x_hbm, idx_v, rows_v, sem0, sem1):
    wid = lax.axis_index("s") * NC + lax.axis_index("c")
    base = wid * CHUNKS_PER_W
    pltpu.sync_copy(tok_hbm.at[wid], idx_v)

    pltpu.async_copy(emb_hbm.at[idx_v.at[0]], rows_v.at[0], sem0).wait()

    def step(c, carry):
        nxt = pltpu.async_copy(emb_hbm.at[idx_v.at[c + 1]],
                               rows_v.at[(c + 1) % 2], sem1)
        pltpu.sync_copy(rows_v.at[c % 2],
                        x_hbm.at[pl.ds((base + c) * CHUNK, CHUNK), pl.ds(0, EMB)])
        nxt.wait()
        return carry

    lax.fori_loop(0, CHUNKS_PER_W - 1, step, 0)
    pltpu.sync_copy(
        rows_v.at[(CHUNKS_PER_W - 1) % 2],
        x_hbm.at[pl.ds((base + CHUNKS_PER_W - 1) * CHUNK, CHUNK), pl.ds(0, EMB)])


def _sc_gather(emb, tok3d):
    mesh = plsc.VectorSubcoreMesh(core_axis_name="c", subcore_axis_name="s")
    fn = pl.kernel(
        _sc_gather_body,
        out_type=jax.ShapeDtypeStruct((TOTAL, XPAD), jnp.float32),
        mesh=mesh,
        scratch_types=[
            pltpu.VMEM((CHUNKS_PER_W, CHUNK), jnp.int32),
            pltpu.VMEM((2, CHUNK, EMB), jnp.float32),
            pltpu.SemaphoreType.DMA,
            pltpu.SemaphoreType.DMA,
        ],
        compiler_params=pltpu.CompilerParams(use_tc_tiling_on_sc=False),
    )
    return fn(emb, tok3d)


def _sigm(x):
    return 0.5 * jnp.tanh(0.5 * x) + 0.5


def _lstm_body(x_ref, W1_ref, U1_ref, b1_ref, W2_ref, U2_ref, b2_ref,
               fc1W_ref, fc1b_ref, fc2Wt_ref, fc2b_ref, out_ref,
               h1_s, c1_s, h2_s, c2_s):
    g = pl.program_id(0)

    @pl.when(g == 0)
    def _():
        h1_s[...] = jnp.zeros_like(h1_s)
        c1_s[...] = jnp.zeros_like(c1_s)
        h2_s[...] = jnp.zeros_like(h2_s)
        c2_s[...] = jnp.zeros_like(c2_s)

    def dot(a, b):
        return lax.dot_general(a, b, (((1,), (0,)), ((), ())),
                               preferred_element_type=jnp.float32)

    def cell(z, c_prev):
        i = _sigm(z[:, 0 * UNITS:1 * UNITS])
        f = _sigm(z[:, 1 * UNITS:2 * UNITS])
        gg = jnp.tanh(z[:, 2 * UNITS:3 * UNITS])
        o = _sigm(z[:, 3 * UNITS:4 * UNITS])
        c_new = f * c_prev + i * gg
        h_new = o * jnp.tanh(c_new)
        return h_new, c_new

    W1 = W1_ref[...]
    U1 = U1_ref[...]
    b1 = b1_ref[...]
    W2 = W2_ref[...]
    U2 = U2_ref[...]
    b2 = b2_ref[...]

    h1, c1 = h1_s[...], c1_s[...]
    h2, c2 = h2_s[...], c2_s[...]
    for j in range(TBLK):
        x_t = x_ref[j, :, 0:EMB]
        h1, c1 = cell(dot(x_t, W1) + dot(h1, U1) + b1, c1)
        h2, c2 = cell(dot(h1, W2) + dot(h2, U2) + b2, c2)
    h1_s[...], c1_s[...] = h1, c1
    h2_s[...], c2_s[...] = h2, c2

    @pl.when(g == NGRID - 1)
    def _():
        d = jnp.tanh(dot(h2, fc1W_ref[...]) + fc1b_ref[...])
        o = jnp.sum(d * fc2Wt_ref[...], axis=1, keepdims=True) + fc2b_ref[...]
        out_ref[...] = _sigm(o)


def _lstm_head(x, W1, U1, b1, W2, U2, b2, fc1_W, fc1_b, fc2_Wt, fc2_b):
    full = lambda shape: pl.BlockSpec(shape, lambda g: (0,) * len(shape))
    return pl.pallas_call(
        _lstm_body,
        grid=(NGRID,),
        in_specs=[
            pl.BlockSpec((TBLK, B, XPAD), lambda g: (g, 0, 0)),
            full((EMB, 4 * UNITS)),
            full((UNITS, 4 * UNITS)),
            full((1, 4 * UNITS)),
            full((UNITS, 4 * UNITS)),
            full((UNITS, 4 * UNITS)),
            full((1, 4 * UNITS)),
            full((UNITS, DENSE)),
            full((1, DENSE)),
            full((1, DENSE)),
            full((1, 1)),
        ],
        out_specs=pl.BlockSpec((B, 1), lambda g: (0, 0)),
        out_shape=jax.ShapeDtypeStruct((B, 1), jnp.float32),
        scratch_shapes=[pltpu.VMEM((B, UNITS), jnp.float32)] * 4,
    )(x, W1, U1, b1, W2, U2, b2, fc1_W, fc1_b, fc2_Wt, fc2_b)


def kernel(tokens, emb, W1, U1, b1, W2, U2, b2, fc1_W, fc1_b, fc2_W, fc2_b):
    tok3d = jnp.transpose(tokens).astype(jnp.int32).reshape(NW, CHUNKS_PER_W, CHUNK)
    x = _sc_gather(emb, tok3d).reshape(L, B, XPAD)
    out = _lstm_head(
        x, W1, U1, b1.reshape(1, -1), W2, U2, b2.reshape(1, -1),
        fc1_W, fc1_b.reshape(1, -1), fc2_W.reshape(1, -1), fc2_b.reshape(1, 1),
    )
    return out


# speedup vs baseline: 1.6490x; 1.0031x over previous
import jax
import jax.numpy as jnp
from jax import lax
from jax.experimental import pallas as pl
from jax.experimental.pallas import tpu as pltpu
from jax.experimental.pallas import tpu_sc as plsc
from jax.experimental import layout as jexp_layout

VOCAB = 1000000
EMB = 64
XPAD = 128
UNITS = 64
DENSE = 250
B = 1024
L = 200

NC = 2
NS = 16
NW = NC * NS
TOTAL = B * L
CHUNK = 128
ROWS_PER_W = TOTAL // NW
CHUNKS_PER_W = ROWS_PER_W // CHUNK

TBLK = 10
NGRID = L // TBLK


def _sc_gather_body(emb_hbm, tok_hbm, x_hbm, idx_v, rows_v, sem0, sem1):
    wid = lax.axis_index("s") * NC + lax.axis_index("c")
    base = wid * CHUNKS_PER_W
    pltpu.sync_copy(tok_hbm.at[wid], idx_v)

    pltpu.async_copy(emb_hbm.at[idx_v.at[0]], rows_v.at[0], sem0).wait()

    def step(c, carry):
        nxt = pltpu.async_copy(emb_hbm.at[idx_v.at[c + 1]],
                               rows_v.at[(c + 1) % 2], sem1)
        pltpu.sync_copy(rows_v.at[c % 2],
                        x_hbm.at[pl.ds((base + c) * CHUNK, CHUNK), pl.ds(0, EMB)])
        nxt.wait()
        return carry

    lax.fori_loop(0, CHUNKS_PER_W - 1, step, 0)
    pltpu.sync_copy(
        rows_v.at[(CHUNKS_PER_W - 1) % 2],
        x_hbm.at[pl.ds((base + CHUNKS_PER_W - 1) * CHUNK, CHUNK), pl.ds(0, EMB)])


def _sc_gather(emb, tok3d):
    mesh = plsc.VectorSubcoreMesh(core_axis_name="c", subcore_axis_name="s")
    fn = pl.kernel(
        _sc_gather_body,
        out_type=jax.ShapeDtypeStruct((TOTAL, XPAD), jnp.float32),
        mesh=mesh,
        scratch_types=[
            pltpu.VMEM((CHUNKS_PER_W, CHUNK), jnp.int32),
            pltpu.VMEM((2, CHUNK, EMB), jnp.float32),
            pltpu.SemaphoreType.DMA,
            pltpu.SemaphoreType.DMA,
        ],
        compiler_params=pltpu.CompilerParams(use_tc_tiling_on_sc=False),
    )
    return fn(emb, tok3d)


def _sigm(x):
    return 0.5 * jnp.tanh(0.5 * x) + 0.5


def _lstm_body(x_ref, W1_ref, U1_ref, b1_ref, W2_ref, U2_ref, b2_ref,
               fc1W_ref, fc1b_ref, fc2Wt_ref, fc2b_ref, out_ref,
               h1_s, c1_s, h2_s, c2_s):
    g = pl.program_id(0)

    @pl.when(g == 0)
    def _():
        h1_s[...] = jnp.zeros_like(h1_s)
        c1_s[...] = jnp.zeros_like(c1_s)
        h2_s[...] = jnp.zeros_like(h2_s)
        c2_s[...] = jnp.zeros_like(c2_s)

    def dot(a, b):
        return lax.dot_general(a, b, (((1,), (0,)), ((), ())),
                               preferred_element_type=jnp.float32)

    def cell(z, c_prev):
        i = _sigm(z[:, 0 * UNITS:1 * UNITS])
        f = _sigm(z[:, 1 * UNITS:2 * UNITS])
        gg = jnp.tanh(z[:, 2 * UNITS:3 * UNITS])
        o = _sigm(z[:, 3 * UNITS:4 * UNITS])
        c_new = f * c_prev + i * gg
        h_new = o * jnp.tanh(c_new)
        return h_new, c_new

    W1 = W1_ref[...]
    U1 = U1_ref[...]
    b1 = b1_ref[...]
    W2 = W2_ref[...]
    U2 = U2_ref[...]
    b2 = b2_ref[...]

    h1, c1 = h1_s[...], c1_s[...]
    h2, c2 = h2_s[...], c2_s[...]
    for j in range(TBLK):
        x_t = x_ref[j, :, 0:EMB]
        h1, c1 = cell(dot(x_t, W1) + dot(h1, U1) + b1, c1)
        h2, c2 = cell(dot(h1, W2) + dot(h2, U2) + b2, c2)
    h1_s[...], c1_s[...] = h1, c1
    h2_s[...], c2_s[...] = h2, c2

    @pl.when(g == NGRID - 1)
    def _():
        d = jnp.tanh(dot(h2, fc1W_ref[...]) + fc1b_ref[...])
        o = jnp.sum(d * fc2Wt_ref[...], axis=1, keepdims=True) + fc2b_ref[...]
        out_ref[...] = _sigm(o)


def _lstm_head(x, W1, U1, b1, W2, U2, b2, fc1_W, fc1_b, fc2_Wt, fc2_b):
    full = lambda shape: pl.BlockSpec(shape, lambda g: (0,) * len(shape))
    return pl.pallas_call(
        _lstm_body,
        grid=(NGRID,),
        in_specs=[
            pl.BlockSpec((TBLK, B, XPAD), lambda g: (g, 0, 0)),
            full((EMB, 4 * UNITS)),
            full((UNITS, 4 * UNITS)),
            full((1, 4 * UNITS)),
            full((UNITS, 4 * UNITS)),
            full((UNITS, 4 * UNITS)),
            full((1, 4 * UNITS)),
            full((UNITS, DENSE)),
            full((1, DENSE)),
            full((1, DENSE)),
            full((1, 1)),
        ],
        out_specs=pl.BlockSpec((B, 1), lambda g: (0, 0)),
        out_shape=jax.ShapeDtypeStruct((B, 1), jnp.float32),
        scratch_shapes=[pltpu.VMEM((B, UNITS), jnp.float32)] * 4,
    )(x, W1, U1, b1, W2, U2, b2, fc1_W, fc1_b, fc2_Wt, fc2_b)


def kernel(tokens, emb, W1, U1, b1, W2, U2, b2, fc1_W, fc1_b, fc2_W, fc2_b):
    tok3d = jnp.transpose(tokens).astype(jnp.int32).reshape(NW, CHUNKS_PER_W, CHUNK)
    x = _sc_gather(emb, tok3d).reshape(L, B, XPAD)
    out = _lstm_head(
        x, W1, U1, b1.reshape(1, -1), W2, U2, b2.reshape(1, -1),
        fc1_W, fc1_b.reshape(1, -1), fc2_W.reshape(1, -1), fc2_b.reshape(1, 1),
    )
    return out


# speedup vs baseline: 1.6776x; 1.0173x over previous
import jax
import jax.numpy as jnp
from jax import lax
from jax.experimental import pallas as pl
from jax.experimental.pallas import tpu as pltpu
from jax.experimental.pallas import tpu_sc as plsc

VOCAB = 1000000
EMB = 64
XPAD = 128
UNITS = 64
DENSE = 250
B = 1024
L = 200

NC = 2
NS = 16
NW = NC * NS
CHUNK = 128

SEG = 5
SEG_STEPS = L // SEG
SEG_ROWS = SEG_STEPS * B
CHUNKS_PER_W = SEG_ROWS // NW // CHUNK

TBLK = 10
NGRID = SEG_STEPS // TBLK


def _sc_gather_body(emb_hbm, tok_hbm, x_hbm, idx_v, rows_v, sem0, sem1):
    wid = lax.axis_index("s") * NC + lax.axis_index("c")
    base = wid * CHUNKS_PER_W
    pltpu.sync_copy(tok_hbm.at[wid], idx_v)

    pltpu.async_copy(emb_hbm.at[idx_v.at[0]], rows_v.at[0], sem0).wait()

    def step(c, carry):
        nxt = pltpu.async_copy(emb_hbm.at[idx_v.at[c + 1]],
                               rows_v.at[(c + 1) % 2], sem1)
        pltpu.sync_copy(rows_v.at[c % 2],
                        x_hbm.at[pl.ds((base + c) * CHUNK, CHUNK), pl.ds(0, EMB)])
        nxt.wait()
        return carry

    lax.fori_loop(0, CHUNKS_PER_W - 1, step, 0)
    pltpu.sync_copy(
        rows_v.at[(CHUNKS_PER_W - 1) % 2],
        x_hbm.at[pl.ds((base + CHUNKS_PER_W - 1) * CHUNK, CHUNK), pl.ds(0, EMB)])


def _sc_gather(emb, tok3d):
    mesh = plsc.VectorSubcoreMesh(core_axis_name="c", subcore_axis_name="s")
    fn = pl.kernel(
        _sc_gather_body,
        out_type=jax.ShapeDtypeStruct((SEG_ROWS, XPAD), jnp.float32),
        mesh=mesh,
        scratch_types=[
            pltpu.VMEM((CHUNKS_PER_W, CHUNK), jnp.int32),
            pltpu.VMEM((2, CHUNK, EMB), jnp.float32),
            pltpu.SemaphoreType.DMA,
            pltpu.SemaphoreType.DMA,
        ],
        compiler_params=pltpu.CompilerParams(use_tc_tiling_on_sc=False),
    )
    return fn(emb, tok3d)


def _sigm(x):
    return 0.5 * jnp.tanh(0.5 * x) + 0.5


def _lstm_body(x_ref, W1_ref, U1_ref, b1_ref, W2_ref, U2_ref, b2_ref,
               fc1W_ref, fc1b_ref, fc2Wt_ref, fc2b_ref,
               h1i, c1i, h2i, c2i,
               h1o, c1o, h2o, c2o, out_ref,
               h1_s, c1_s, h2_s, c2_s):
    g = pl.program_id(0)

    @pl.when(g == 0)
    def _():
        h1_s[...] = h1i[...]
        c1_s[...] = c1i[...]
        h2_s[...] = h2i[...]
        c2_s[...] = c2i[...]

    def dot(a, b):
        return lax.dot_general(a, b, (((1,), (0,)), ((), ())),
                               preferred_element_type=jnp.float32)

    def cell(z, c_prev):
        i = _sigm(z[:, 0 * UNITS:1 * UNITS])
        f = _sigm(z[:, 1 * UNITS:2 * UNITS])
        gg = jnp.tanh(z[:, 2 * UNITS:3 * UNITS])
        o = _sigm(z[:, 3 * UNITS:4 * UNITS])
        c_new = f * c_prev + i * gg
        h_new = o * jnp.tanh(c_new)
        return h_new, c_new

    W1 = W1_ref[...]
    U1 = U1_ref[...]
    b1 = b1_ref[...]
    W2 = W2_ref[...]
    U2 = U2_ref[...]
    b2 = b2_ref[...]

    h1, c1 = h1_s[...], c1_s[...]
    h2, c2 = h2_s[...], c2_s[...]
    for j in range(TBLK):
        x_t = x_ref[j, :, 0:EMB]
        h1, c1 = cell(dot(x_t, W1) + dot(h1, U1) + b1, c1)
        h2, c2 = cell(dot(h1, W2) + dot(h2, U2) + b2, c2)
    h1_s[...], c1_s[...] = h1, c1
    h2_s[...], c2_s[...] = h2, c2

    @pl.when(g == NGRID - 1)
    def _():
        h1o[...], c1o[...] = h1, c1
        h2o[...], c2o[...] = h2, c2
        d = jnp.tanh(dot(h2, fc1W_ref[...]) + fc1b_ref[...])
        o = jnp.sum(d * fc2Wt_ref[...], axis=1, keepdims=True) + fc2b_ref[...]
        out_ref[...] = _sigm(o)


def _lstm_seg(x, W1, U1, b1, W2, U2, b2, fc1_W, fc1_b, fc2_Wt, fc2_b,
              h1, c1, h2, c2):
    full = lambda shape: pl.BlockSpec(shape, lambda g: (0,) * len(shape))
    st = jax.ShapeDtypeStruct((B, UNITS), jnp.float32)
    return pl.pallas_call(
        _lstm_body,
        grid=(NGRID,),
        in_specs=[
            pl.BlockSpec((TBLK, B, XPAD), lambda g: (g, 0, 0)),
            full((EMB, 4 * UNITS)),
            full((UNITS, 4 * UNITS)),
            full((1, 4 * UNITS)),
            full((UNITS, 4 * UNITS)),
            full((UNITS, 4 * UNITS)),
            full((1, 4 * UNITS)),
            full((UNITS, DENSE)),
            full((1, DENSE)),
            full((1, DENSE)),
            full((1, 1)),
            full((B, UNITS)),
            full((B, UNITS)),
            full((B, UNITS)),
            full((B, UNITS)),
        ],
        out_specs=[
            full((B, UNITS)),
            full((B, UNITS)),
            full((B, UNITS)),
            full((B, UNITS)),
            pl.BlockSpec((B, 1), lambda g: (0, 0)),
        ],
        out_shape=[st, st, st, st, jax.ShapeDtypeStruct((B, 1), jnp.float32)],
        scratch_shapes=[pltpu.VMEM((B, UNITS), jnp.float32)] * 4,
    )(x, W1, U1, b1, W2, U2, b2, fc1_W, fc1_b, fc2_Wt, fc2_b, h1, c1, h2, c2)


def kernel(tokens, emb, W1, U1, b1, W2, U2, b2, fc1_W, fc1_b, fc2_W, fc2_b):
    tokT = jnp.transpose(tokens).astype(jnp.int32)
    b1r, b2r = b1.reshape(1, -1), b2.reshape(1, -1)
    fc1b = fc1_b.reshape(1, -1)
    fc2Wt = fc2_W.reshape(1, -1)
    fc2b = fc2_b.reshape(1, 1)
    z = jnp.zeros((B, UNITS), jnp.float32)
    h1 = c1 = h2 = c2 = z
    out = None
    for s in range(SEG):
        tok3d = tokT[s * SEG_STEPS:(s + 1) * SEG_STEPS].reshape(
            NW, CHUNKS_PER_W, CHUNK)
        x = _sc_gather(emb, tok3d).reshape(SEG_STEPS, B, XPAD)
        h1, c1, h2, c2, out = _lstm_seg(
            x, W1, U1, b1r, W2, U2, b2r, fc1_W, fc1b, fc2Wt, fc2b,
            h1, c1, h2, c2)
    return out
